# baseline (speedup 1.0000x reference)
import functools

import jax
import jax.numpy as jnp
from jax import lax
from jax.experimental import pallas as pl
from jax.experimental.pallas import tpu as pltpu
from jax.experimental.pallas import tpu_sc as plsc

B = 16384
D2 = 64

_info = plsc.get_sparse_core_info()
NC = _info.num_cores
NS = _info.num_subcores
NW = NC * NS
BPW = B // NW

_mesh = plsc.VectorSubcoreMesh(core_axis_name="c", subcore_axis_name="s")


@functools.partial(
    pl.kernel,
    mesh=_mesh,
    compiler_params=pltpu.CompilerParams(
        use_tc_tiling_on_sc=False, needs_layout_passes=False),
    out_type=jax.ShapeDtypeStruct((B, 2, D2), jnp.float32),
    scratch_types=[
        pltpu.VMEM((BPW, 2), jnp.int32),
        pltpu.VMEM((BPW,), jnp.int32),
        pltpu.VMEM((BPW,), jnp.int32),
        pltpu.VMEM((BPW, D2), jnp.float32),
        pltpu.VMEM((BPW, D2), jnp.float32),
        pltpu.SemaphoreType.DMA,
        pltpu.SemaphoreType.DMA,
    ],
)
def _emb_kernel(pos_hbm, wx_hbm, wy_hbm, out_hbm,
                pos_v, idxx_v, idxy_v, rowsx_v, rowsy_v, semx, semy):
    wid = lax.axis_index("s") * NC + lax.axis_index("c")
    base = wid * BPW
    pltpu.sync_copy(pos_hbm.at[pl.ds(base, BPW)], pos_v)
    lane = lax.iota(jnp.int32, 16)
    zero = lane * 0
    one = zero + 1
    for i in range(BPW // 16):
        row = lane + (16 * i)
        idxx_v[pl.ds(16 * i, 16)] = plsc.load_gather(pos_v, [row, zero])
        idxy_v[pl.ds(16 * i, 16)] = plsc.load_gather(pos_v, [row, one])
    cx = pltpu.async_copy(wx_hbm.at[idxx_v], rowsx_v, semx)
    cy = pltpu.async_copy(wy_hbm.at[idxy_v], rowsy_v, semy)
    cx.wait()
    pltpu.sync_copy(rowsx_v, out_hbm.at[pl.ds(base, BPW), 0])
    cy.wait()
    pltpu.sync_copy(rowsy_v, out_hbm.at[pl.ds(base, BPW), 1])


def kernel(positions, Wx, Wy):
    out = _emb_kernel(positions.astype(jnp.int32), Wx, Wy)
    return out.reshape(B, 2 * D2)


# speedup vs baseline: 1.5167x; 1.5167x over previous
import functools

import jax
import jax.numpy as jnp
from jax import lax
from jax.experimental import pallas as pl
from jax.experimental.pallas import tpu as pltpu
from jax.experimental.pallas import tpu_sc as plsc

B = 16384
D2 = 64

_info = plsc.get_sparse_core_info()
NC = _info.num_cores
NS = _info.num_subcores
NW = NC * NS
BPW = B // NW

_mesh = plsc.VectorSubcoreMesh(core_axis_name="c", subcore_axis_name="s")


@functools.partial(
    pl.kernel,
    mesh=_mesh,
    compiler_params=pltpu.CompilerParams(
        use_tc_tiling_on_sc=False, needs_layout_passes=False),
    out_type=jax.ShapeDtypeStruct((B, 2 * D2), jnp.float32),
    scratch_types=[
        pltpu.VMEM((BPW,), jnp.int32),
        pltpu.VMEM((BPW,), jnp.int32),
        pltpu.VMEM((BPW, D2), jnp.float32),
        pltpu.VMEM((BPW, D2), jnp.float32),
        pltpu.SemaphoreType.DMA,
        pltpu.SemaphoreType.DMA,
    ],
)
def _emb_kernel(posf_hbm, wx_hbm, wy_hbm, out_hbm,
                idxx_v, idxy_v, rowsx_v, rowsy_v, semx, semy):
    wid = lax.axis_index("s") * NC + lax.axis_index("c")
    base = wid * BPW
    pltpu.sync_copy(posf_hbm.at[pl.ds(base, BPW)], idxx_v)
    pltpu.sync_copy(posf_hbm.at[pl.ds(B + base, BPW)], idxy_v)
    cx = pltpu.async_copy(wx_hbm.at[idxx_v], rowsx_v, semx)
    cy = pltpu.async_copy(wy_hbm.at[idxy_v], rowsy_v, semy)
    cx.wait()
    pltpu.sync_copy(rowsx_v, out_hbm.at[pl.ds(base, BPW), pl.ds(0, D2)])
    cy.wait()
    pltpu.sync_copy(rowsy_v, out_hbm.at[pl.ds(base, BPW), pl.ds(D2, D2)])


def kernel(positions, Wx, Wy):
    posf = positions.astype(jnp.int32).T.reshape(2 * B)
    return _emb_kernel(posf, Wx, Wy)
